# baseline (speedup 1.0000x reference)
import functools

import jax
import jax.numpy as jnp
from jax import lax
from jax.experimental import pallas as pl
from jax.experimental.pallas import tpu as pltpu
from jax.experimental.pallas import tpu_sc as plsc

B = 1024
L = 50
D = 64
NC = 2
NS = 16
NW = NC * NS
BPW = B // NW
RPW = BPW * L
CHUNK = 128
NCHUNK = (RPW + CHUNK - 1) // CHUNK
RPAD = NCHUNK * CHUNK

@functools.cache
def _get_mesh():
    return plsc.VectorSubcoreMesh(
        core_axis_name="c", subcore_axis_name="s", num_cores=NC, num_subcores=NS
    )


def _pool_body(hist_hbm, table_hbm, out_hbm, idx_v, rows_v, acc_v, isem, gsem):
    wid = lax.axis_index("s") * NC + lax.axis_index("c")
    base = wid * RPW

    pltpu.async_copy(hist_hbm.at[wid], idx_v.at[...], isem).wait()

    copies = []
    for j in range(NCHUNK):
        copies.append(
            pltpu.async_copy(
                table_hbm.at[idx_v.at[j]],
                rows_v.at[pl.ds(j * CHUNK, CHUNK)],
                gsem,
            )
        )
    for c in copies:
        c.wait()

    def ev_body(e, carry):
        r0 = e * L
        accs = [rows_v[r0, pl.ds(j * 16, 16)] for j in range(4)]
        for r in range(1, L):
            for j in range(4):
                accs[j] = accs[j] + rows_v[r0 + r, pl.ds(j * 16, 16)]
        for j in range(4):
            acc_v[e, pl.ds(j * 16, 16)] = accs[j]
        return carry

    lax.fori_loop(0, BPW, ev_body, 0, unroll=False)

    pltpu.sync_copy(acc_v.at[...], out_hbm.at[pl.ds(wid * BPW, BPW)])


@functools.cache
def _get_pool():
    return pl.kernel(
        _pool_body,
        out_type=jax.ShapeDtypeStruct((B, D), jnp.float32),
        mesh=_get_mesh(),
        scratch_types=[
            pltpu.VMEM((NCHUNK, CHUNK), jnp.int32),
            pltpu.VMEM((RPAD, D), jnp.float32),
            pltpu.VMEM((BPW, D), jnp.float32),
            pltpu.SemaphoreType.DMA,
            pltpu.SemaphoreType.DMA,
        ],
        compiler_params=pltpu.CompilerParams(use_tc_tiling_on_sc=False),
    )


def _linear_body(his_ref, wt_ref, b_ref, out_ref):
    out_ref[...] = (
        jnp.dot(his_ref[...], wt_ref[...], preferred_element_type=jnp.float32)
        + b_ref[...]
    )


def kernel(entities, history, entities_emb, W, b):
    del entities
    hist = history.astype(jnp.int32).reshape(B, L)
    hist = hist.reshape(NW, RPW)
    hist = jnp.pad(hist, ((0, 0), (0, RPAD - RPW))).reshape(NW, NCHUNK, CHUNK)
    his = _get_pool()(hist, entities_emb)
    out = pl.pallas_call(
        _linear_body,
        out_shape=jax.ShapeDtypeStruct((B, D), jnp.float32),
    )(his, W.T, b.reshape(1, D))
    return out
